# initial kernel scaffold (unmeasured)
import jax
import jax.numpy as jnp
from jax import lax
from jax.experimental import pallas as pl
from jax.experimental.pallas import tpu as pltpu

N_DEV = 16


def _silu(y):
    return y * (1.0 / (1.0 + jnp.exp(-y)))


def kernel(x, w_mat):
    m_per, k = x.shape
    n_per = w_mat.shape[1]

    def body(x_ref, w_ref, out_ref, comm_ref, send_sems, recv_sems):
        my = lax.axis_index("i")
        left = lax.rem(my + N_DEV - 1, N_DEV)
        right = lax.rem(my + 1, N_DEV)

        barrier_sem = pltpu.get_barrier_semaphore()
        for nbr in [left, right]:
            pl.semaphore_signal(
                barrier_sem, inc=1,
                device_id=(nbr,), device_id_type=pl.DeviceIdType.MESH,
            )
        pl.semaphore_wait(barrier_sem, 2)

        comm_ref[0, :, :] = x_ref[:, :]

        y = jnp.dot(x_ref[:, :], w_ref[:, :], preferred_element_type=jnp.float32)
        out_ref[pl.ds(my * m_per, m_per), :] = _silu(y)

        for h in range(N_DEV - 1):
            rdma = pltpu.make_async_remote_copy(
                src_ref=comm_ref.at[h],
                dst_ref=comm_ref.at[h + 1],
                send_sem=send_sems.at[h],
                recv_sem=recv_sems.at[h],
                device_id=(right,),
                device_id_type=pl.DeviceIdType.MESH,
            )
            rdma.start()
            rdma.wait()

            origin = lax.rem(my + N_DEV - 1 - h, N_DEV)
            y = jnp.dot(
                comm_ref[h + 1, :, :], w_ref[:, :],
                preferred_element_type=jnp.float32,
            )
            out_ref[pl.ds(origin * m_per, m_per), :] = _silu(y)

    return pl.pallas_call(
        body,
        out_shape=jax.ShapeDtypeStruct((N_DEV * m_per, n_per), jnp.float32),
        in_specs=[
            pl.BlockSpec(memory_space=pltpu.VMEM),
            pl.BlockSpec(memory_space=pltpu.VMEM),
        ],
        out_specs=pl.BlockSpec(memory_space=pltpu.VMEM),
        scratch_shapes=[
            pltpu.VMEM((N_DEV, m_per, k), x.dtype),
            pltpu.SemaphoreType.DMA((N_DEV - 1,)),
            pltpu.SemaphoreType.DMA((N_DEV - 1,)),
        ],
        compiler_params=pltpu.CompilerParams(collective_id=0),
    )(x, w_mat)


# baseline (device time: 413181 ns/iter reference)
import jax
import jax.numpy as jnp
from jax import lax
from jax.experimental import pallas as pl
from jax.experimental.pallas import tpu as pltpu

N_DEV = 16


def _silu(y):
    return y * (1.0 / (1.0 + jnp.exp(-y)))


def kernel(x, w_mat):
    m_per, k = x.shape
    n_per = w_mat.shape[1]
    x = x.astype(jnp.bfloat16)
    w_mat = w_mat.astype(jnp.bfloat16)

    def body(x_ref, w_ref, out_ref, comm_ref, send_sems, recv_sems):
        my = lax.axis_index("i")
        left = lax.rem(my + N_DEV - 1, N_DEV)
        right = lax.rem(my + 1, N_DEV)

        barrier_sem = pltpu.get_barrier_semaphore()
        for nbr in [left, right]:
            pl.semaphore_signal(
                barrier_sem, inc=1,
                device_id=(nbr,), device_id_type=pl.DeviceIdType.MESH,
            )
        pl.semaphore_wait(barrier_sem, 2)

        comm_ref[0, :, :] = x_ref[:, :]

        y = jnp.dot(x_ref[:, :], w_ref[:, :], preferred_element_type=jnp.float32)
        out_ref[pl.ds(my * m_per, m_per), :] = _silu(y)

        for h in range(N_DEV - 1):
            rdma = pltpu.make_async_remote_copy(
                src_ref=comm_ref.at[h],
                dst_ref=comm_ref.at[h + 1],
                send_sem=send_sems.at[h],
                recv_sem=recv_sems.at[h],
                device_id=(right,),
                device_id_type=pl.DeviceIdType.MESH,
            )
            rdma.start()
            rdma.wait()

            origin = lax.rem(my + N_DEV - 1 - h, N_DEV)
            y = jnp.dot(
                comm_ref[h + 1, :, :], w_ref[:, :],
                preferred_element_type=jnp.float32,
            )
            out_ref[pl.ds(origin * m_per, m_per), :] = _silu(y)

    return pl.pallas_call(
        body,
        out_shape=jax.ShapeDtypeStruct((N_DEV * m_per, n_per), jnp.float32),
        in_specs=[
            pl.BlockSpec(memory_space=pltpu.VMEM),
            pl.BlockSpec(memory_space=pltpu.VMEM),
        ],
        out_specs=pl.BlockSpec(memory_space=pltpu.VMEM),
        scratch_shapes=[
            pltpu.VMEM((N_DEV, m_per, k), x.dtype),
            pltpu.SemaphoreType.DMA((N_DEV - 1,)),
            pltpu.SemaphoreType.DMA((N_DEV - 1,)),
        ],
        compiler_params=pltpu.CompilerParams(
            collective_id=0,
            vmem_limit_bytes=63 * 1024 * 1024,
        ),
    )(x, w_mat)


# device time: 223959 ns/iter; 1.8449x vs baseline; 1.8449x over previous
import jax
import jax.numpy as jnp
from jax import lax
from jax.experimental import pallas as pl
from jax.experimental.pallas import tpu as pltpu

N_DEV = 16


def _silu(y):
    return y * (1.0 / (1.0 + jnp.exp(-y)))


def kernel(x, w_mat):
    m_per, k = x.shape
    n_per = w_mat.shape[1]
    half = m_per // 2
    x = x.astype(jnp.bfloat16)
    w_mat = w_mat.astype(jnp.bfloat16)

    def body(x_ref, w_ref, out_ref, cw_buf, ccw_buf,
             cw_send, cw_recv, ccw_send, ccw_recv):
        my = lax.axis_index("i")
        left = lax.rem(my + N_DEV - 1, N_DEV)
        right = lax.rem(my + 1, N_DEV)

        barrier_sem = pltpu.get_barrier_semaphore()
        for nbr in [left, right]:
            pl.semaphore_signal(
                barrier_sem, inc=1,
                device_id=(nbr,), device_id_type=pl.DeviceIdType.MESH,
            )
        pl.semaphore_wait(barrier_sem, 2)

        cw_buf[0, :, :] = x_ref[:half, :]
        ccw_buf[0, :, :] = x_ref[half:, :]

        cw_rdma = [
            pltpu.make_async_remote_copy(
                src_ref=cw_buf.at[h], dst_ref=cw_buf.at[h + 1],
                send_sem=cw_send.at[h], recv_sem=cw_recv.at[h],
                device_id=(right,), device_id_type=pl.DeviceIdType.MESH,
            )
            for h in range(N_DEV - 1)
        ]
        ccw_rdma = [
            pltpu.make_async_remote_copy(
                src_ref=ccw_buf.at[h], dst_ref=ccw_buf.at[h + 1],
                send_sem=ccw_send.at[h], recv_sem=ccw_recv.at[h],
                device_id=(left,), device_id_type=pl.DeviceIdType.MESH,
            )
            for h in range(N_DEV - 1)
        ]

        cw_rdma[0].start()
        ccw_rdma[0].start()

        y = jnp.dot(x_ref[:, :], w_ref[:, :], preferred_element_type=jnp.float32)
        out_ref[pl.ds(my * m_per, m_per), :] = _silu(y)

        for h in range(N_DEV - 1):
            cw_rdma[h].wait_recv()
            if h + 1 < N_DEV - 1:
                cw_rdma[h + 1].start()
            ccw_rdma[h].wait_recv()
            if h + 1 < N_DEV - 1:
                ccw_rdma[h + 1].start()

            cw_origin = lax.rem(my + N_DEV - 1 - h, N_DEV)
            y = jnp.dot(cw_buf[h + 1, :, :], w_ref[:, :],
                        preferred_element_type=jnp.float32)
            out_ref[pl.ds(cw_origin * m_per, half), :] = _silu(y)

            ccw_origin = lax.rem(my + h + 1, N_DEV)
            y = jnp.dot(ccw_buf[h + 1, :, :], w_ref[:, :],
                        preferred_element_type=jnp.float32)
            out_ref[pl.ds(ccw_origin * m_per + half, half), :] = _silu(y)

        for h in range(N_DEV - 1):
            cw_rdma[h].wait_send()
            ccw_rdma[h].wait_send()

    return pl.pallas_call(
        body,
        out_shape=jax.ShapeDtypeStruct((N_DEV * m_per, n_per), jnp.float32),
        in_specs=[
            pl.BlockSpec(memory_space=pltpu.VMEM),
            pl.BlockSpec(memory_space=pltpu.VMEM),
        ],
        out_specs=pl.BlockSpec(memory_space=pltpu.VMEM),
        scratch_shapes=[
            pltpu.VMEM((N_DEV, half, k), jnp.bfloat16),
            pltpu.VMEM((N_DEV, half, k), jnp.bfloat16),
            pltpu.SemaphoreType.DMA((N_DEV - 1,)),
            pltpu.SemaphoreType.DMA((N_DEV - 1,)),
            pltpu.SemaphoreType.DMA((N_DEV - 1,)),
            pltpu.SemaphoreType.DMA((N_DEV - 1,)),
        ],
        compiler_params=pltpu.CompilerParams(
            collective_id=0,
            vmem_limit_bytes=63 * 1024 * 1024,
        ),
    )(x, w_mat)


# device time: 201584 ns/iter; 2.0497x vs baseline; 1.1110x over previous
import jax
import jax.numpy as jnp
from jax import lax
from jax.experimental import pallas as pl
from jax.experimental.pallas import tpu as pltpu

N_DEV = 16
NSEG = 4


def _silu(y):
    return y * (1.0 / (1.0 + jnp.exp(-y)))


def kernel(x, w_mat):
    m_per, k = x.shape
    n_per = w_mat.shape[1]
    half = m_per // 2
    seg = half // NSEG
    x = x.astype(jnp.bfloat16)
    w_mat = w_mat.astype(jnp.bfloat16)

    def body(x_ref, w_ref, out_ref, cw_buf, ccw_buf,
             cw_send, cw_recv, ccw_send, ccw_recv):
        my = lax.axis_index("i")
        left = lax.rem(my + N_DEV - 1, N_DEV)
        right = lax.rem(my + 1, N_DEV)

        barrier_sem = pltpu.get_barrier_semaphore()
        for nbr in [left, right]:
            pl.semaphore_signal(
                barrier_sem, inc=1,
                device_id=(nbr,), device_id_type=pl.DeviceIdType.MESH,
            )
        pl.semaphore_wait(barrier_sem, 2)

        cw_buf[0, :, :] = x_ref[:half, :]
        ccw_buf[0, :, :] = x_ref[half:, :]

        def make(buf, send_sems, recv_sems, h, s, dst):
            return pltpu.make_async_remote_copy(
                src_ref=buf.at[h, pl.ds(s * seg, seg)],
                dst_ref=buf.at[h + 1, pl.ds(s * seg, seg)],
                send_sem=send_sems.at[h, s],
                recv_sem=recv_sems.at[h, s],
                device_id=(dst,), device_id_type=pl.DeviceIdType.MESH,
            )

        cw_rdma = [[make(cw_buf, cw_send, cw_recv, h, s, right)
                    for s in range(NSEG)] for h in range(N_DEV - 1)]
        ccw_rdma = [[make(ccw_buf, ccw_send, ccw_recv, h, s, left)
                     for s in range(NSEG)] for h in range(N_DEV - 1)]

        for s in range(NSEG):
            cw_rdma[0][s].start()
            ccw_rdma[0][s].start()

        y = jnp.dot(x_ref[:, :], w_ref[:, :], preferred_element_type=jnp.float32)
        out_ref[pl.ds(my * m_per, m_per), :] = _silu(y)

        for h in range(N_DEV - 1):
            for s in range(NSEG):
                cw_rdma[h][s].wait_recv()
                if h + 1 < N_DEV - 1:
                    cw_rdma[h + 1][s].start()
                ccw_rdma[h][s].wait_recv()
                if h + 1 < N_DEV - 1:
                    ccw_rdma[h + 1][s].start()

            cw_origin = lax.rem(my + N_DEV - 1 - h, N_DEV)
            y = jnp.dot(cw_buf[h + 1, :, :], w_ref[:, :],
                        preferred_element_type=jnp.float32)
            out_ref[pl.ds(cw_origin * m_per, half), :] = _silu(y)

            ccw_origin = lax.rem(my + h + 1, N_DEV)
            y = jnp.dot(ccw_buf[h + 1, :, :], w_ref[:, :],
                        preferred_element_type=jnp.float32)
            out_ref[pl.ds(ccw_origin * m_per + half, half), :] = _silu(y)

        for h in range(N_DEV - 1):
            for s in range(NSEG):
                cw_rdma[h][s].wait_send()
                ccw_rdma[h][s].wait_send()

    return pl.pallas_call(
        body,
        out_shape=jax.ShapeDtypeStruct((N_DEV * m_per, n_per), jnp.float32),
        in_specs=[
            pl.BlockSpec(memory_space=pltpu.VMEM),
            pl.BlockSpec(memory_space=pltpu.VMEM),
        ],
        out_specs=pl.BlockSpec(memory_space=pltpu.VMEM),
        scratch_shapes=[
            pltpu.VMEM((N_DEV, half, k), jnp.bfloat16),
            pltpu.VMEM((N_DEV, half, k), jnp.bfloat16),
            pltpu.SemaphoreType.DMA((N_DEV - 1, NSEG)),
            pltpu.SemaphoreType.DMA((N_DEV - 1, NSEG)),
            pltpu.SemaphoreType.DMA((N_DEV - 1, NSEG)),
            pltpu.SemaphoreType.DMA((N_DEV - 1, NSEG)),
        ],
        compiler_params=pltpu.CompilerParams(
            collective_id=0,
            vmem_limit_bytes=63 * 1024 * 1024,
        ),
    )(x, w_mat)
